# baseline (speedup 1.0000x reference)
import functools

import jax
import jax.numpy as jnp
from jax import lax
from jax.experimental import pallas as pl
from jax.experimental.pallas import tpu as pltpu
from jax.experimental.pallas import tpu_sc as plsc

EMB = 32
BLK = 2048


def _body(idx_ref, u_tab, i_tab, w1u_ref, w1i_ref, b1_ref, w2_ref, b2_ref,
          w3_ref, b3_ref, o_ref, u_buf, i_buf, usem, isem):
    g = pl.program_id(0)
    base = g * BLK

    def issue(k, _):
        uidx = idx_ref[0, base + k]
        iidx = idx_ref[1, base + k]
        pltpu.make_async_copy(u_tab.at[pl.ds(uidx, 1)],
                              u_buf.at[pl.ds(k, 1)], usem).start()
        pltpu.make_async_copy(i_tab.at[pl.ds(iidx, 1)],
                              i_buf.at[pl.ds(k, 1)], isem).start()
        return 0

    lax.fori_loop(0, BLK, issue, 0, unroll=8)

    def drain(k, _):
        pltpu.make_async_copy(u_tab.at[pl.ds(0, 1)],
                              u_buf.at[pl.ds(k, 1)], usem).wait()
        pltpu.make_async_copy(i_tab.at[pl.ds(0, 1)],
                              i_buf.at[pl.ds(k, 1)], isem).wait()
        return 0

    lax.fori_loop(0, BLK, drain, 0, unroll=8)

    x1 = jnp.dot(u_buf[...], w1u_ref[...], preferred_element_type=jnp.float32)
    x1 += jnp.dot(i_buf[...], w1i_ref[...], preferred_element_type=jnp.float32)
    x1 = jnp.maximum(x1 + b1_ref[...], 0.0)
    x2 = jnp.dot(x1, w2_ref[...], preferred_element_type=jnp.float32)
    x2 = jnp.maximum(x2 + b2_ref[...], 0.0)
    logits = jnp.dot(x2, w3_ref[...], preferred_element_type=jnp.float32)
    logits = logits + b3_ref[...]
    m = jnp.max(logits, axis=1, keepdims=True)
    e = jnp.exp(logits - m)
    o_ref[...] = e / jnp.sum(e, axis=1, keepdims=True)


def kernel(user, item, user_table, item_table, W1, b1, W2, b2, W3, b3):
    B = user.shape[0]
    idx = jnp.stack([user.astype(jnp.int32), item.astype(jnp.int32)])
    W1uT = W1[:, :EMB].T
    W1iT = W1[:, EMB:].T
    full = lambda shape: pl.BlockSpec(shape, lambda i, s: (0, 0))
    grid_spec = pltpu.PrefetchScalarGridSpec(
        num_scalar_prefetch=1,
        grid=(B // BLK,),
        in_specs=[
            pl.BlockSpec(memory_space=pl.ANY),
            pl.BlockSpec(memory_space=pl.ANY),
            full((EMB, 64)),
            full((EMB, 64)),
            full((1, 64)),
            full((64, 32)),
            full((1, 32)),
            full((32, 5)),
            full((1, 5)),
        ],
        out_specs=pl.BlockSpec((BLK, 5), lambda i, s: (i, 0)),
        scratch_shapes=[
            pltpu.VMEM((BLK, EMB), jnp.float32),
            pltpu.VMEM((BLK, EMB), jnp.float32),
            pltpu.SemaphoreType.DMA,
            pltpu.SemaphoreType.DMA,
        ],
    )
    return pl.pallas_call(
        _body,
        grid_spec=grid_spec,
        out_shape=jax.ShapeDtypeStruct((B, 5), jnp.float32),
        compiler_params=pltpu.CompilerParams(
            dimension_semantics=("parallel",)),
    )(idx, user_table, item_table, W1uT, W1iT, b1.reshape(1, -1),
      W2.T, b2.reshape(1, -1), W3.T, b3.reshape(1, -1))


# speedup vs baseline: 1.2256x; 1.2256x over previous
import functools

import jax
import jax.numpy as jnp
from jax import lax
from jax.experimental import pallas as pl
from jax.experimental.pallas import tpu as pltpu
from jax.experimental.pallas import tpu_sc as plsc

EMB = 32
NC = 2
NS = 16
NW = NC * NS
BLK = 2048


def _gather_sc(user_r, item_r, user_table, item_table, B):
    b_per_w = B // NW
    n_chunks = -(-b_per_w // 256)
    chunk = b_per_w // n_chunks
    mesh = plsc.VectorSubcoreMesh(core_axis_name="c", subcore_axis_name="s")

    @functools.partial(
        pl.kernel,
        mesh=mesh,
        out_type=jax.ShapeDtypeStruct((B, 128), jnp.float32),
        scratch_types=[
            pltpu.SMEM((b_per_w,), jnp.int32),
            pltpu.SMEM((b_per_w,), jnp.int32),
            pltpu.VMEM_SHARED((NS, b_per_w), jnp.int32),
            pltpu.VMEM_SHARED((NS, b_per_w), jnp.int32),
            pltpu.VMEM((chunk, EMB), jnp.float32),
            pltpu.VMEM((chunk, EMB), jnp.float32),
            pltpu.VMEM((chunk, 128), jnp.float32),
            pltpu.SemaphoreType.DMA,
            pltpu.SemaphoreType.DMA,
        ],
    )
    def gather_kernel(u_idx_hbm, i_idx_hbm, u_tab, i_tab, out,
                      uidx_s, iidx_s, ush_v, ish_v, urows_v, irows_v,
                      dense_v, usem, isem):
        sid = lax.axis_index("s")
        wid = sid * NC + lax.axis_index("c")
        base = wid * b_per_w
        pltpu.sync_copy(u_idx_hbm.at[wid], ush_v.at[sid])
        pltpu.sync_copy(i_idx_hbm.at[wid], ish_v.at[sid])
        pltpu.sync_copy(ush_v.at[sid], uidx_s)
        pltpu.sync_copy(ish_v.at[sid], iidx_s)

        for ch in range(n_chunks):
            off = ch * chunk

            @plsc.parallel_loop(0, chunk, unroll=8)
            def _(k, off=off):
                pltpu.async_copy(u_tab.at[pl.ds(uidx_s[off + k], 1)],
                                 urows_v.at[pl.ds(k, 1)], usem)
                pltpu.async_copy(i_tab.at[pl.ds(iidx_s[off + k], 1)],
                                 irows_v.at[pl.ds(k, 1)], isem)

            pltpu.make_async_copy(u_tab.at[pl.ds(0, chunk)],
                                  urows_v, usem).wait()
            pltpu.make_async_copy(i_tab.at[pl.ds(0, chunk)],
                                  irows_v, isem).wait()

            @plsc.parallel_loop(0, chunk, unroll=4)
            def _(k):
                for lo in (0, 16):
                    us = (pl.ds(k, 1), pl.ds(lo, 16))
                    dense_v.at[*us][...] = urows_v.at[*us][...]
                    id_ = (pl.ds(k, 1), pl.ds(EMB + lo, 16))
                    dense_v.at[*id_][...] = irows_v.at[*us][...]

            pltpu.sync_copy(dense_v, out.at[pl.ds(base + off, chunk)])

    return gather_kernel(user_r, item_r, user_table, item_table)


def _mlp_body(x_ref, w1u_ref, w1i_ref, b1_ref, w2_ref, b2_ref,
              w3_ref, b3_ref, o_ref):
    u = x_ref[:, :EMB]
    i = x_ref[:, EMB:2 * EMB]
    x1 = jnp.dot(u, w1u_ref[...], preferred_element_type=jnp.float32)
    x1 += jnp.dot(i, w1i_ref[...], preferred_element_type=jnp.float32)
    x1 = jnp.maximum(x1 + b1_ref[...], 0.0)
    x2 = jnp.dot(x1, w2_ref[...], preferred_element_type=jnp.float32)
    x2 = jnp.maximum(x2 + b2_ref[...], 0.0)
    logits = jnp.dot(x2, w3_ref[...], preferred_element_type=jnp.float32)
    logits = logits + b3_ref[...]
    m = jnp.max(logits, axis=1, keepdims=True)
    e = jnp.exp(logits - m)
    o_ref[...] = e / jnp.sum(e, axis=1, keepdims=True)


def _mlp_tc(x_emb, W1uT, W1iT, b1, W2T, b2, W3T, b3):
    B = x_emb.shape[0]
    n_out = W3T.shape[1]
    full = lambda shape: pl.BlockSpec(shape, lambda i: (0, 0))
    return pl.pallas_call(
        _mlp_body,
        grid=(B // BLK,),
        in_specs=[
            pl.BlockSpec((BLK, 128), lambda i: (i, 0)),
            full(W1uT.shape),
            full(W1iT.shape),
            full(b1.shape),
            full(W2T.shape),
            full(b2.shape),
            full(W3T.shape),
            full(b3.shape),
        ],
        out_specs=pl.BlockSpec((BLK, n_out), lambda i: (i, 0)),
        out_shape=jax.ShapeDtypeStruct((B, n_out), jnp.float32),
    )(x_emb, W1uT, W1iT, b1, W2T, b2, W3T, b3)


def kernel(user, item, user_table, item_table, W1, b1, W2, b2, W3, b3):
    B = user.shape[0]
    b_per_w = B // NW
    user_r = user.astype(jnp.int32).reshape(NW, b_per_w)
    item_r = item.astype(jnp.int32).reshape(NW, b_per_w)
    x_emb = _gather_sc(user_r, item_r, user_table, item_table, B)
    W1uT = W1[:, :EMB].T
    W1iT = W1[:, EMB:].T
    return _mlp_tc(x_emb, W1uT, W1iT, b1.reshape(1, -1),
                   W2.T, b2.reshape(1, -1), W3.T, b3.reshape(1, -1))
